# baseline (speedup 1.0000x reference)
import jax
import jax.numpy as jnp
from jax.experimental import pallas as pl
from jax.experimental.pallas import tpu as pltpu

IN_DIM = 16
HIDDEN = 32
NUM_HEADS = 4
HEAD_DIM = HIDDEN // NUM_HEADS
NUM_LAYERS = 2
FFN_DIM = HIDDEN * NUM_HEADS
N_NODES = 8
LN_EPS = 1e-5
PACK_W = 128
_NEG = -1e30

SUB_G = 16
SUBR = SUB_G * N_NODES
HL = NUM_HEADS * SUBR


def _pack_layout():
    layout = {}
    mats = [('emb_w', (IN_DIM, HIDDEN))]
    vecs = [('emb_b', HIDDEN)]
    for l in range(NUM_LAYERS):
        mats += [(f'wqkv{l}', (HIDDEN, 3 * HIDDEN)),
                 (f'wo{l}',   (HIDDEN, HIDDEN)),
                 (f'w1{l}',   (HIDDEN, FFN_DIM)),
                 (f'w2{l}',   (FFN_DIM, HIDDEN))]
        vecs += [(f'bqkv{l}', 3 * HIDDEN), (f'bo{l}', HIDDEN),
                 (f'ln1_g{l}', HIDDEN), (f'ln1_b{l}', HIDDEN),
                 (f'b1{l}', FFN_DIM), (f'b2{l}', HIDDEN),
                 (f'ln2_g{l}', HIDDEN), (f'ln2_b{l}', HIDDEN)]
    vecs += [('fc_wT', HIDDEN), ('fc_b', 1)]
    rows = 0
    for name, (nr, nc) in mats:
        layout[name] = (rows, 0, (nr, nc))
        rows += -(-nr // 8) * 8
    cur_row, cur_col = rows, 0
    for name, nc in vecs:
        if cur_col + nc > PACK_W:
            cur_row += 1
            cur_col = 0
        layout[name] = (cur_row, cur_col, (1, nc))
        cur_col += nc
    total = -(-(cur_row + 1) // 8) * 8
    return layout, total


_LAYOUT, _TOTAL_ROWS = _pack_layout()


def _graphormer_block(feat_ref, p_ref, cn_ref, hm_ref, oh_ref, sel_ref, out_ref):
    GB, N = out_ref.shape
    R = feat_ref.shape[0]
    f32 = jnp.float32

    def get(name):
        r, c, (nr, nc) = _LAYOUT[name]
        return p_ref[r:r + nr, c:c + nc]

    def layer_norm(x, g, b):
        mu = jnp.mean(x, axis=-1, keepdims=True)
        var = jnp.mean((x - mu) ** 2, axis=-1, keepdims=True)
        return (x - mu) * jax.lax.rsqrt(var + LN_EPS) * g + b

    cross_neg = cn_ref[...]
    head_mask = hm_ref[...]
    node_oh = oh_ref[...]
    sel = sel_ref[...]

    h = jnp.dot(feat_ref[...], get('emb_w'),
                preferred_element_type=f32) + get('emb_b')

    for l in range(NUM_LAYERS):
        qkv = jnp.dot(h, get(f'wqkv{l}'),
                      preferred_element_type=f32) + get(f'bqkv{l}')
        ctx_parts = []
        for b in range(R // SUBR):
            rows = slice(b * SUBR, (b + 1) * SUBR)
            qb = qkv[rows, :HIDDEN]
            kvb = qkv[rows, HIDDEN:]
            kv_rep = jnp.concatenate(
                [kvb * head_mask[hh * SUBR:hh * SUBR + 1, :]
                 for hh in range(NUM_HEADS)], axis=0)
            k_rep = kv_rep[:, :HIDDEN]
            v_rep = kv_rep[:, HIDDEN:]
            s = jax.lax.dot_general(qb, k_rep, (((1,), (1,)), ((), ())),
                                    preferred_element_type=f32) + cross_neg
            ps, dens = [], []
            for hh in range(NUM_HEADS):
                sh = s[:, hh * SUBR:(hh + 1) * SUBR]
                mh = jnp.max(sh, axis=-1, keepdims=True)
                ph = jnp.exp(sh - mh)
                ps.append(ph)
                dens.append(jnp.sum(ph, axis=-1, keepdims=True))
            p = jnp.concatenate(ps, axis=1)
            ctxb = jnp.dot(p, v_rep, preferred_element_type=f32)
            den = jnp.concatenate(
                [jnp.broadcast_to(d, (SUBR, HEAD_DIM)) for d in dens], axis=1)
            ctx_parts.append(ctxb * pl.reciprocal(den, approx=True))
        ctx = jnp.concatenate(ctx_parts, axis=0)

        attn = jnp.dot(ctx, get(f'wo{l}'),
                       preferred_element_type=f32) + get(f'bo{l}')
        h = layer_norm(h + attn, get(f'ln1_g{l}'), get(f'ln1_b{l}'))
        f = jnp.dot(h, get(f'w1{l}'),
                    preferred_element_type=f32) + get(f'b1{l}')
        f = jnp.maximum(f, 0.0)
        f = jnp.dot(f, get(f'w2{l}'),
                    preferred_element_type=f32) + get(f'b2{l}')
        h = layer_norm(h + f, get(f'ln2_g{l}'), get(f'ln2_b{l}'))

    logit = jnp.sum(h * get('fc_wT'), axis=-1, keepdims=True) + get('fc_b')
    slab = jnp.broadcast_to(logit, (R, N)) * node_oh
    out_ref[...] = jnp.dot(sel, slab, preferred_element_type=f32)


def kernel(features, packed_params):
    total_nodes, in_dim = features.shape
    assert in_dim == IN_DIM
    B = total_nodes // N_NODES
    gb = next(g for g in (256, 128, 64, 32, 16) if B % g == 0)
    R = gb * N_NODES
    f32 = jnp.float32
    rowg = jax.lax.broadcasted_iota(jnp.int32, (SUBR, HL), 0) // N_NODES
    laneg = (jax.lax.broadcasted_iota(jnp.int32, (SUBR, HL), 1)
             % SUBR) // N_NODES
    cross_neg = jnp.where(rowg == laneg, 0.0, _NEG)
    reph = jax.lax.broadcasted_iota(jnp.int32, (HL, 2 * HIDDEN), 0) // SUBR
    repl = (jax.lax.broadcasted_iota(jnp.int32, (HL, 2 * HIDDEN), 1)
            % HIDDEN) // HEAD_DIM
    head_mask = (reph == repl).astype(f32)
    node_oh = (jax.lax.broadcasted_iota(jnp.int32, (R, N_NODES), 0) % N_NODES ==
               jax.lax.broadcasted_iota(jnp.int32, (R, N_NODES), 1)).astype(f32)
    sel = (jax.lax.broadcasted_iota(jnp.int32, (gb, R), 1) // N_NODES ==
           jax.lax.broadcasted_iota(jnp.int32, (gb, R), 0)).astype(f32)
    const_spec = lambda shape: pl.BlockSpec(shape, lambda i: (0, 0))
    return pl.pallas_call(
        _graphormer_block,
        out_shape=jax.ShapeDtypeStruct((B, N_NODES), jnp.float32),
        grid=(B // gb,),
        in_specs=[
            pl.BlockSpec((R, IN_DIM), lambda i: (i, 0)),
            pl.BlockSpec((_TOTAL_ROWS, PACK_W), lambda i: (0, 0)),
            const_spec((SUBR, HL)),
            const_spec((HL, 2 * HIDDEN)),
            const_spec((R, N_NODES)),
            const_spec((gb, R)),
        ],
        out_specs=pl.BlockSpec((gb, N_NODES), lambda i: (i, 0)),
        compiler_params=pltpu.CompilerParams(
            dimension_semantics=("parallel",)),
    )(features, packed_params, cross_neg, head_mask, node_oh, sel)


# speedup vs baseline: 1.0049x; 1.0049x over previous
import jax
import jax.numpy as jnp
from jax.experimental import pallas as pl
from jax.experimental.pallas import tpu as pltpu

IN_DIM = 16
HIDDEN = 32
NUM_HEADS = 4
HEAD_DIM = HIDDEN // NUM_HEADS
NUM_LAYERS = 2
FFN_DIM = HIDDEN * NUM_HEADS
N_NODES = 8
LN_EPS = 1e-5
PACK_W = 128
_NEG = -1e30

SUB_G = 16
SUBR = SUB_G * N_NODES
HL = NUM_HEADS * SUBR


def _pack_layout():
    layout = {}
    mats = [('emb_w', (IN_DIM, HIDDEN))]
    vecs = [('emb_b', HIDDEN)]
    for l in range(NUM_LAYERS):
        mats += [(f'wqkv{l}', (HIDDEN, 3 * HIDDEN)),
                 (f'wo{l}',   (HIDDEN, HIDDEN)),
                 (f'w1{l}',   (HIDDEN, FFN_DIM)),
                 (f'w2{l}',   (FFN_DIM, HIDDEN))]
        vecs += [(f'bqkv{l}', 3 * HIDDEN), (f'bo{l}', HIDDEN),
                 (f'ln1_g{l}', HIDDEN), (f'ln1_b{l}', HIDDEN),
                 (f'b1{l}', FFN_DIM), (f'b2{l}', HIDDEN),
                 (f'ln2_g{l}', HIDDEN), (f'ln2_b{l}', HIDDEN)]
    vecs += [('fc_wT', HIDDEN), ('fc_b', 1)]
    rows = 0
    for name, (nr, nc) in mats:
        layout[name] = (rows, 0, (nr, nc))
        rows += -(-nr // 8) * 8
    cur_row, cur_col = rows, 0
    for name, nc in vecs:
        if cur_col + nc > PACK_W:
            cur_row += 1
            cur_col = 0
        layout[name] = (cur_row, cur_col, (1, nc))
        cur_col += nc
    total = -(-(cur_row + 1) // 8) * 8
    return layout, total


_LAYOUT, _TOTAL_ROWS = _pack_layout()


def _graphormer_block(feat_ref, p_ref, out_ref):
    GB, N = out_ref.shape
    R = feat_ref.shape[0]
    f32 = jnp.float32

    def get(name):
        r, c, (nr, nc) = _LAYOUT[name]
        return p_ref[r:r + nr, c:c + nc]

    def layer_norm(x, g, b):
        mu = jnp.mean(x, axis=-1, keepdims=True)
        var = jnp.mean((x - mu) ** 2, axis=-1, keepdims=True)
        return (x - mu) * jax.lax.rsqrt(var + LN_EPS) * g + b

    rowg = jax.lax.broadcasted_iota(jnp.int32, (SUBR, HL), 0) // N
    laneg = (jax.lax.broadcasted_iota(jnp.int32, (SUBR, HL), 1) % SUBR) // N
    cross_neg = jnp.where(rowg == laneg, 0.0, _NEG)
    reph = jax.lax.broadcasted_iota(jnp.int32, (HL, 2 * HIDDEN), 0) // SUBR
    repl = (jax.lax.broadcasted_iota(jnp.int32, (HL, 2 * HIDDEN), 1)
            % HIDDEN) // HEAD_DIM
    head_mask = (reph == repl).astype(f32)
    den_mask = head_mask[:, :HIDDEN]
    node_oh = (jax.lax.broadcasted_iota(jnp.int32, (R, N), 0) % N ==
               jax.lax.broadcasted_iota(jnp.int32, (R, N), 1)).astype(f32)
    sel = (jax.lax.broadcasted_iota(jnp.int32, (GB, R), 1) // N ==
           jax.lax.broadcasted_iota(jnp.int32, (GB, R), 0)).astype(f32)

    h = jnp.dot(feat_ref[...], get('emb_w'),
                preferred_element_type=f32) + get('emb_b')

    for l in range(NUM_LAYERS):
        qkv = jnp.dot(h, get(f'wqkv{l}'),
                      preferred_element_type=f32) + get(f'bqkv{l}')
        ctx_parts = []
        for b in range(R // SUBR):
            rows = slice(b * SUBR, (b + 1) * SUBR)
            qb = qkv[rows, :HIDDEN]
            kvb = qkv[rows, HIDDEN:]
            kv_rep = jnp.concatenate(
                [kvb * head_mask[hh * SUBR:hh * SUBR + 1, :]
                 for hh in range(NUM_HEADS)], axis=0)
            k_rep = kv_rep[:, :HIDDEN]
            v_rep = kv_rep[:, HIDDEN:]
            s = jax.lax.dot_general(qb, k_rep, (((1,), (1,)), ((), ())),
                                    preferred_element_type=f32) + cross_neg
            ps, dens = [], []
            for hh in range(NUM_HEADS):
                sh = s[:, hh * SUBR:(hh + 1) * SUBR]
                mh = jnp.max(sh, axis=-1, keepdims=True)
                ph = jnp.exp(sh - mh)
                ps.append(ph)
                dens.append(jnp.sum(ph, axis=-1, keepdims=True))
            p = jnp.concatenate(ps, axis=1)
            ctxb = jnp.dot(p, v_rep, preferred_element_type=f32)
            den = jnp.concatenate(
                [jnp.broadcast_to(d, (SUBR, HEAD_DIM)) for d in dens], axis=1)
            ctx_parts.append(ctxb * pl.reciprocal(den, approx=True))
        ctx = jnp.concatenate(ctx_parts, axis=0)

        attn = jnp.dot(ctx, get(f'wo{l}'),
                       preferred_element_type=f32) + get(f'bo{l}')
        h = layer_norm(h + attn, get(f'ln1_g{l}'), get(f'ln1_b{l}'))
        f = jnp.dot(h, get(f'w1{l}'),
                    preferred_element_type=f32) + get(f'b1{l}')
        f = jnp.maximum(f, 0.0)
        f = jnp.dot(f, get(f'w2{l}'),
                    preferred_element_type=f32) + get(f'b2{l}')
        h = layer_norm(h + f, get(f'ln2_g{l}'), get(f'ln2_b{l}'))

    logit = jnp.sum(h * get('fc_wT'), axis=-1, keepdims=True) + get('fc_b')
    slab = jnp.broadcast_to(logit, (R, N)) * node_oh
    out_ref[...] = jnp.dot(sel, slab, preferred_element_type=f32)


def kernel(features, packed_params):
    total_nodes, in_dim = features.shape
    assert in_dim == IN_DIM
    B = total_nodes // N_NODES
    gb = next(g for g in (256, 128, 64, 32, 16) if B % g == 0)
    return pl.pallas_call(
        _graphormer_block,
        out_shape=jax.ShapeDtypeStruct((B, N_NODES), jnp.float32),
        grid=(B // gb,),
        in_specs=[
            pl.BlockSpec((gb * N_NODES, IN_DIM), lambda i: (i, 0)),
            pl.BlockSpec((_TOTAL_ROWS, PACK_W), lambda i: (0, 0)),
        ],
        out_specs=pl.BlockSpec((gb, N_NODES), lambda i: (i, 0)),
        compiler_params=pltpu.CompilerParams(
            dimension_semantics=("parallel",)),
    )(features, packed_params)
